# Initial kernel scaffold; baseline (speedup 1.0000x reference)
import jax
import jax.numpy as jnp
from jax.experimental import pallas as pl


def kernel(x, edge_index, head_batched, rel_batched, tail_batched, time_batched, gcn_W, gcn_b, rel_emb, norm_emb):
    raise NotImplementedError("write your pallas kernel here")


# speedup vs baseline: 3.2353x; 3.2353x over previous
import functools

import jax
import jax.numpy as jnp
from jax import lax
from jax.experimental import pallas as pl
from jax.experimental.pallas import tpu as pltpu
from jax.experimental.pallas import tpu_sc as plsc

N_NODES = 10000
N_EDGES = 320000
DIM = 128
BATCH = 4096

NC = 2
NS = 16
NW = NC * NS

K = 128
CH_PER_W = 80
NCHUNK = CH_PER_W * NW
E_PAD = NCHUNK * K

NPAD = 10112
ROWS_PER_TILE = NPAD // NS

_mesh = plsc.VectorSubcoreMesh(core_axis_name="c", subcore_axis_name="s")


@functools.partial(
    pl.kernel,
    mesh=_mesh,
    out_type=jax.ShapeDtypeStruct((NC, NPAD, DIM), jnp.float32),
    scratch_types=[
        pltpu.VMEM((CH_PER_W, K), jnp.int32),
        pltpu.VMEM((CH_PER_W, K), jnp.int32),
        pltpu.VMEM((K, DIM), jnp.float32),
        pltpu.VMEM_SHARED((NPAD, DIM), jnp.float32),
        pltpu.SemaphoreType.DMA,
    ],
)
def _sc_segment_sum(table, srcs, dsts, zeros, out, src_v, dst_v, rows_v, acc, sem):
    c = lax.axis_index("c")
    s = lax.axis_index("s")
    wid = c * NS + s

    pltpu.sync_copy(zeros.at[pl.ds(s * ROWS_PER_TILE, ROWS_PER_TILE)],
                    acc.at[pl.ds(s * ROWS_PER_TILE, ROWS_PER_TILE)])
    pltpu.sync_copy(srcs.at[pl.ds(wid * CH_PER_W, CH_PER_W)], src_v)
    pltpu.sync_copy(dsts.at[pl.ds(wid * CH_PER_W, CH_PER_W)], dst_v)
    plsc.subcore_barrier()

    def body(j, carry):
        pltpu.async_copy(table.at[src_v.at[j]], rows_v, sem).wait()
        pltpu.sync_copy(rows_v, acc.at[dst_v.at[j]], add=True)
        return carry

    lax.fori_loop(0, CH_PER_W, body, 0)
    plsc.subcore_barrier()
    pltpu.sync_copy(acc.at[pl.ds(s * ROWS_PER_TILE, ROWS_PER_TILE)],
                    out.at[c, pl.ds(s * ROWS_PER_TILE, ROWS_PER_TILE)])


B_PER_W = BATCH // NW


@functools.partial(
    pl.kernel,
    mesh=_mesh,
    out_type=[jax.ShapeDtypeStruct((BATCH, DIM), jnp.float32) for _ in range(4)],
    scratch_types=[
        pltpu.VMEM((B_PER_W,), jnp.int32),
        pltpu.VMEM((B_PER_W, DIM), jnp.float32),
        pltpu.SemaphoreType.DMA,
    ],
)
def _sc_gather(feat, rel_emb, norm_emb, heads, rels, tails, times,
               out_h, out_r, out_t, out_nv, idx_v, buf, sem):
    c = lax.axis_index("c")
    s = lax.axis_index("s")
    wid = c * NS + s
    base = wid * B_PER_W
    for idx_hbm, tbl, dst in ((heads, feat, out_h), (rels, rel_emb, out_r),
                              (tails, feat, out_t), (times, norm_emb, out_nv)):
        pltpu.sync_copy(idx_hbm.at[pl.ds(base, B_PER_W)], idx_v)
        pltpu.async_copy(tbl.at[idx_v], buf, sem).wait()
        pltpu.sync_copy(buf, dst.at[pl.ds(base, B_PER_W)])


ACT_BLK = 1000


def _act_body(p_ref, w_ref, b_ref, o_ref):
    agg = p_ref[0] + p_ref[1]
    y = lax.dot_general(agg, w_ref[...], (((1,), (1,)), ((), ())),
                        preferred_element_type=jnp.float32)
    o_ref[...] = jnp.tanh(y + b_ref[...])


_tc_act = pl.pallas_call(
    _act_body,
    grid=(N_NODES // ACT_BLK,),
    in_specs=[
        pl.BlockSpec((NC, ACT_BLK, DIM), lambda i: (0, i, 0)),
        pl.BlockSpec((DIM, DIM), lambda i: (0, 0)),
        pl.BlockSpec((1, DIM), lambda i: (0, 0)),
    ],
    out_specs=pl.BlockSpec((ACT_BLK, DIM), lambda i: (i, 0)),
    out_shape=jax.ShapeDtypeStruct((N_NODES, DIM), jnp.float32),
)

SCORE_BLK = 1024


def _l2n(e, eps=1e-12):
    n = jnp.sqrt(jnp.sum(e * e, axis=-1, keepdims=True))
    return e / jnp.maximum(n, eps)


def _score_body(h_ref, r_ref, t_ref, nv_ref, o_ref):
    nvn = _l2n(nv_ref[...])

    def proj(e):
        return e - jnp.sum(nvn * e, axis=-1, keepdims=True) * nvn

    h = _l2n(proj(h_ref[...]))
    r = _l2n(proj(r_ref[...]))
    t = _l2n(proj(t_ref[...]))
    d = h + r - t
    o_ref[...] = jnp.sqrt(jnp.sum(d * d, axis=-1, keepdims=True))


_tc_score = pl.pallas_call(
    _score_body,
    grid=(BATCH // SCORE_BLK,),
    in_specs=[pl.BlockSpec((SCORE_BLK, DIM), lambda i: (i, 0)) for _ in range(4)],
    out_specs=pl.BlockSpec((SCORE_BLK, 1), lambda i: (i, 0)),
    out_shape=jax.ShapeDtypeStruct((BATCH, 1), jnp.float32),
)


def kernel(x, edge_index, head_batched, rel_batched, tail_batched, time_batched,
           gcn_W, gcn_b, rel_emb, norm_emb):
    src = edge_index[0]
    dst = edge_index[1]
    pad = E_PAD - N_EDGES
    srcs = jnp.concatenate([src, jnp.zeros((pad,), jnp.int32)]).reshape(NCHUNK, K)
    dsts = jnp.concatenate([dst, jnp.full((pad,), N_NODES, jnp.int32)]).reshape(NCHUNK, K)
    zeros = jnp.zeros((NPAD, DIM), jnp.float32)
    b2 = gcn_b.reshape(1, DIM)

    p1 = _sc_segment_sum(x, srcs, dsts, zeros)
    h1 = _tc_act(p1, gcn_W, b2)
    p2 = _sc_segment_sum(h1, srcs, dsts, zeros)
    feat = _tc_act(p2, gcn_W, b2)

    h, r, t, nv = _sc_gather(feat, rel_emb, norm_emb, head_batched,
                             rel_batched, tail_batched, time_batched)
    return _tc_score(h, r, t, nv).reshape(-1)


# speedup vs baseline: 3.6396x; 1.1249x over previous
import functools

import jax
import jax.numpy as jnp
from jax import lax
from jax.experimental import pallas as pl
from jax.experimental.pallas import tpu as pltpu
from jax.experimental.pallas import tpu_sc as plsc

N_NODES = 10000
N_EDGES = 320000
DIM = 128
BATCH = 4096

NC = 2
NS = 16
NW = NC * NS

K = 128
CH_PER_W = 80
NCHUNK = CH_PER_W * NW
E_PAD = NCHUNK * K

NPAD = 10112
ROWS_PER_TILE = NPAD // NS

_mesh = plsc.VectorSubcoreMesh(core_axis_name="c", subcore_axis_name="s")


NBUF = 2
NHALF = 2
M = CH_PER_W // NHALF


@functools.partial(
    pl.kernel,
    mesh=_mesh,
    out_type=jax.ShapeDtypeStruct((NC, NPAD, DIM), jnp.float32),
    scratch_types=[
        pltpu.VMEM((M, K), jnp.int32),
        pltpu.VMEM((M, K), jnp.int32),
        pltpu.VMEM((NBUF, K, DIM), jnp.float32),
        pltpu.VMEM_SHARED((NPAD, DIM), jnp.float32),
        pltpu.SemaphoreType.DMA((NBUF,)),
        pltpu.SemaphoreType.DMA((NBUF,)),
    ],
)
def _sc_segment_sum(table, srcs, dsts, zeros, out, src_v, dst_v, rows_v, acc,
                    gsem, ssem):
    c = lax.axis_index("c")
    s = lax.axis_index("s")
    wid = c * NS + s

    pltpu.sync_copy(zeros.at[pl.ds(s * ROWS_PER_TILE, ROWS_PER_TILE)],
                    acc.at[pl.ds(s * ROWS_PER_TILE, ROWS_PER_TILE)])
    plsc.subcore_barrier()

    for h in range(NHALF):
        pltpu.sync_copy(srcs.at[pl.ds(wid * CH_PER_W + h * M, M)], src_v)
        pltpu.sync_copy(dsts.at[pl.ds(wid * CH_PER_W + h * M, M)], dst_v)

        def step(j, carry):
            b = lax.rem(j, NBUF)

            @pl.when(j < M)
            def _gather():
                @pl.when(j >= NBUF)
                def _free():
                    pltpu.make_async_copy(rows_v.at[b],
                                          acc.at[dst_v.at[j - NBUF]],
                                          ssem.at[b]).wait()

                pltpu.async_copy(table.at[src_v.at[j]], rows_v.at[b],
                                 gsem.at[b])

            @pl.when(j >= 1)
            def _scatter():
                b2 = lax.rem(j - 1, NBUF)
                pltpu.make_async_copy(table.at[src_v.at[j - 1]],
                                      rows_v.at[b2], gsem.at[b2]).wait()
                pltpu.async_copy(rows_v.at[b2], acc.at[dst_v.at[j - 1]],
                                 ssem.at[b2], add=True)

            return carry

        lax.fori_loop(0, M + 1, step, 0)
        for bb in range(NBUF):
            pltpu.make_async_copy(rows_v.at[bb], acc.at[dst_v.at[0]],
                                  ssem.at[bb]).wait()

    plsc.subcore_barrier()
    pltpu.sync_copy(acc.at[pl.ds(s * ROWS_PER_TILE, ROWS_PER_TILE)],
                    out.at[c, pl.ds(s * ROWS_PER_TILE, ROWS_PER_TILE)])


B_PER_W = BATCH // NW


@functools.partial(
    pl.kernel,
    mesh=_mesh,
    out_type=[jax.ShapeDtypeStruct((BATCH, DIM), jnp.float32) for _ in range(4)],
    scratch_types=[
        pltpu.VMEM((B_PER_W,), jnp.int32),
        pltpu.VMEM((B_PER_W, DIM), jnp.float32),
        pltpu.SemaphoreType.DMA,
    ],
)
def _sc_gather(feat, rel_emb, norm_emb, heads, rels, tails, times,
               out_h, out_r, out_t, out_nv, idx_v, buf, sem):
    c = lax.axis_index("c")
    s = lax.axis_index("s")
    wid = c * NS + s
    base = wid * B_PER_W
    for idx_hbm, tbl, dst in ((heads, feat, out_h), (rels, rel_emb, out_r),
                              (tails, feat, out_t), (times, norm_emb, out_nv)):
        pltpu.sync_copy(idx_hbm.at[pl.ds(base, B_PER_W)], idx_v)
        pltpu.async_copy(tbl.at[idx_v], buf, sem).wait()
        pltpu.sync_copy(buf, dst.at[pl.ds(base, B_PER_W)])


ACT_BLK = 1000


def _act_body(p_ref, w_ref, b_ref, o_ref):
    agg = p_ref[0] + p_ref[1]
    y = lax.dot_general(agg, w_ref[...], (((1,), (1,)), ((), ())),
                        preferred_element_type=jnp.float32)
    o_ref[...] = jnp.tanh(y + b_ref[...])


_tc_act = pl.pallas_call(
    _act_body,
    grid=(N_NODES // ACT_BLK,),
    in_specs=[
        pl.BlockSpec((NC, ACT_BLK, DIM), lambda i: (0, i, 0)),
        pl.BlockSpec((DIM, DIM), lambda i: (0, 0)),
        pl.BlockSpec((1, DIM), lambda i: (0, 0)),
    ],
    out_specs=pl.BlockSpec((ACT_BLK, DIM), lambda i: (i, 0)),
    out_shape=jax.ShapeDtypeStruct((N_NODES, DIM), jnp.float32),
)

SCORE_BLK = 1024


def _l2n(e, eps=1e-12):
    n = jnp.sqrt(jnp.sum(e * e, axis=-1, keepdims=True))
    return e / jnp.maximum(n, eps)


def _score_body(h_ref, r_ref, t_ref, nv_ref, o_ref):
    nvn = _l2n(nv_ref[...])

    def proj(e):
        return e - jnp.sum(nvn * e, axis=-1, keepdims=True) * nvn

    h = _l2n(proj(h_ref[...]))
    r = _l2n(proj(r_ref[...]))
    t = _l2n(proj(t_ref[...]))
    d = h + r - t
    o_ref[...] = jnp.sqrt(jnp.sum(d * d, axis=-1, keepdims=True))


_tc_score = pl.pallas_call(
    _score_body,
    grid=(BATCH // SCORE_BLK,),
    in_specs=[pl.BlockSpec((SCORE_BLK, DIM), lambda i: (i, 0)) for _ in range(4)],
    out_specs=pl.BlockSpec((SCORE_BLK, 1), lambda i: (i, 0)),
    out_shape=jax.ShapeDtypeStruct((BATCH, 1), jnp.float32),
)


def kernel(x, edge_index, head_batched, rel_batched, tail_batched, time_batched,
           gcn_W, gcn_b, rel_emb, norm_emb):
    src = edge_index[0]
    dst = edge_index[1]
    pad = E_PAD - N_EDGES
    srcs = jnp.concatenate([src, jnp.zeros((pad,), jnp.int32)]).reshape(NCHUNK, K)
    dsts = jnp.concatenate([dst, jnp.full((pad,), N_NODES, jnp.int32)]).reshape(NCHUNK, K)
    zeros = jnp.zeros((NPAD, DIM), jnp.float32)
    b2 = gcn_b.reshape(1, DIM)

    p1 = _sc_segment_sum(x, srcs, dsts, zeros)
    h1 = _tc_act(p1, gcn_W, b2)
    p2 = _sc_segment_sum(h1, srcs, dsts, zeros)
    feat = _tc_act(p2, gcn_W, b2)

    h, r, t, nv = _sc_gather(feat, rel_emb, norm_emb, head_batched,
                             rel_batched, tail_batched, time_batched)
    return _tc_score(h, r, t, nv).reshape(-1)


# speedup vs baseline: 11.4499x; 3.1459x over previous
import functools

import jax
import jax.numpy as jnp
from jax import lax
from jax.experimental import pallas as pl
from jax.experimental.pallas import tpu as pltpu
from jax.experimental.pallas import tpu_sc as plsc

N_NODES = 10000
N_EDGES = 320000
DIM = 128
BATCH = 4096

NC = 2
NS = 16
NW = NC * NS

K = 128
CH_PER_W = 80
NCHUNK = CH_PER_W * NW
E_PAD = NCHUNK * K

NPAD = 10112
ROWS_PER_TILE = NPAD // NS

_mesh = plsc.VectorSubcoreMesh(core_axis_name="c", subcore_axis_name="s")


NBUF = 2
NHALF = 2
M = CH_PER_W // NHALF


@functools.partial(
    pl.kernel,
    mesh=_mesh,
    out_type=jax.ShapeDtypeStruct((NC, NPAD, DIM), jnp.float32),
    scratch_types=[
        pltpu.VMEM((M, K), jnp.int32),
        pltpu.VMEM((M, K), jnp.int32),
        pltpu.VMEM((NBUF, K, DIM), jnp.float32),
        pltpu.VMEM_SHARED((NPAD, DIM), jnp.float32),
        pltpu.SemaphoreType.DMA((NBUF,)),
        pltpu.SemaphoreType.DMA((NBUF,)),
    ],
)
def _sc_segment_sum(table, srcs, dsts, zeros, out, src_v, dst_v, rows_v, acc,
                    gsem, ssem):
    c = lax.axis_index("c")
    s = lax.axis_index("s")
    wid = c * NS + s

    pltpu.sync_copy(zeros.at[pl.ds(s * ROWS_PER_TILE, ROWS_PER_TILE)],
                    acc.at[pl.ds(s * ROWS_PER_TILE, ROWS_PER_TILE)])
    plsc.subcore_barrier()

    for h in range(NHALF):
        pltpu.sync_copy(srcs.at[pl.ds(wid * CH_PER_W + h * M, M)], src_v)
        pltpu.sync_copy(dsts.at[pl.ds(wid * CH_PER_W + h * M, M)], dst_v)

        def step(j, carry):
            b = lax.rem(j, NBUF)

            @pl.when(j < M)
            def _gather():
                @pl.when(j >= NBUF)
                def _free():
                    pltpu.make_async_copy(rows_v.at[b],
                                          acc.at[dst_v.at[j - NBUF]],
                                          ssem.at[b]).wait()

                pltpu.async_copy(table.at[src_v.at[j]], rows_v.at[b],
                                 gsem.at[b])

            @pl.when(j >= 1)
            def _scatter():
                b2 = lax.rem(j - 1, NBUF)
                pltpu.make_async_copy(table.at[src_v.at[j - 1]],
                                      rows_v.at[b2], gsem.at[b2]).wait()
                pltpu.async_copy(rows_v.at[b2], acc.at[dst_v.at[j - 1]],
                                 ssem.at[b2], add=True)

            return carry

        lax.fori_loop(0, M + 1, step, 0)
        for bb in range(NBUF):
            pltpu.make_async_copy(rows_v.at[bb], acc.at[dst_v.at[0]],
                                  ssem.at[bb]).wait()

    plsc.subcore_barrier()
    pltpu.sync_copy(acc.at[pl.ds(s * ROWS_PER_TILE, ROWS_PER_TILE)],
                    out.at[c, pl.ds(s * ROWS_PER_TILE, ROWS_PER_TILE)])


B_PER_W = BATCH // NW


@functools.partial(
    pl.kernel,
    mesh=_mesh,
    out_type=[jax.ShapeDtypeStruct((BATCH, DIM), jnp.float32) for _ in range(4)],
    scratch_types=[
        pltpu.VMEM((B_PER_W,), jnp.int32),
        pltpu.VMEM((B_PER_W, DIM), jnp.float32),
        pltpu.SemaphoreType.DMA,
    ],
)
def _sc_gather(feat, rel_emb, norm_emb, heads, rels, tails, times,
               out_h, out_r, out_t, out_nv, idx_v, buf, sem):
    c = lax.axis_index("c")
    s = lax.axis_index("s")
    wid = c * NS + s
    base = wid * B_PER_W
    for idx_hbm, tbl, dst in ((heads, feat, out_h), (rels, rel_emb, out_r),
                              (tails, feat, out_t), (times, norm_emb, out_nv)):
        pltpu.sync_copy(idx_hbm.at[pl.ds(base, B_PER_W)], idx_v)
        pltpu.async_copy(tbl.at[idx_v], buf, sem).wait()
        pltpu.sync_copy(buf, dst.at[pl.ds(base, B_PER_W)])


ACT_BLK = 1000


def _act_body(p_ref, w_ref, b_ref, o_ref):
    agg = p_ref[0] + p_ref[1]
    y = lax.dot_general(agg, w_ref[...], (((1,), (1,)), ((), ())),
                        preferred_element_type=jnp.float32)
    o_ref[...] = jnp.tanh(y + b_ref[...])


_tc_act = pl.pallas_call(
    _act_body,
    grid=(N_NODES // ACT_BLK,),
    in_specs=[
        pl.BlockSpec((NC, ACT_BLK, DIM), lambda i: (0, i, 0)),
        pl.BlockSpec((DIM, DIM), lambda i: (0, 0)),
        pl.BlockSpec((1, DIM), lambda i: (0, 0)),
    ],
    out_specs=pl.BlockSpec((ACT_BLK, DIM), lambda i: (i, 0)),
    out_shape=jax.ShapeDtypeStruct((N_NODES, DIM), jnp.float32),
)

SCORE_BLK = 1024


def _l2n(e, eps=1e-12):
    n = jnp.sqrt(jnp.sum(e * e, axis=-1, keepdims=True))
    return e / jnp.maximum(n, eps)


def _score_body(h_ref, r_ref, t_ref, nv_ref, o_ref):
    nvn = _l2n(nv_ref[...])

    def proj(e):
        return e - jnp.sum(nvn * e, axis=-1, keepdims=True) * nvn

    h = _l2n(proj(h_ref[...]))
    r = _l2n(proj(r_ref[...]))
    t = _l2n(proj(t_ref[...]))
    d = h + r - t
    o_ref[...] = jnp.sqrt(jnp.sum(d * d, axis=-1, keepdims=True))


_tc_score = pl.pallas_call(
    _score_body,
    grid=(BATCH // SCORE_BLK,),
    in_specs=[pl.BlockSpec((SCORE_BLK, DIM), lambda i: (i, 0)) for _ in range(4)],
    out_specs=pl.BlockSpec((SCORE_BLK, 1), lambda i: (i, 0)),
    out_shape=jax.ShapeDtypeStruct((BATCH, 1), jnp.float32),
)


def kernel(x, edge_index, head_batched, rel_batched, tail_batched, time_batched,
           gcn_W, gcn_b, rel_emb, norm_emb):
    src = edge_index[0]
    dst = edge_index[1]
    pad = E_PAD - N_EDGES
    pad_i = jnp.arange(pad, dtype=jnp.int32)
    src_pad = pad_i % N_NODES
    dst_pad = N_NODES + pad_i % (NPAD - N_NODES)
    def chunked(a, a_pad):
        a2 = jnp.concatenate([a, a_pad]).reshape(CH_PER_W, NW, K)
        return a2.swapaxes(0, 1).reshape(NCHUNK, K)

    srcs = chunked(src, src_pad)
    dsts = chunked(dst, dst_pad)
    zeros = jnp.zeros((NPAD, DIM), jnp.float32)
    b2 = gcn_b.reshape(1, DIM)

    p1 = _sc_segment_sum(x, srcs, dsts, zeros)
    h1 = _tc_act(p1, gcn_W, b2)
    p2 = _sc_segment_sum(h1, srcs, dsts, zeros)
    feat = _tc_act(p2, gcn_W, b2)

    h, r, t, nv = _sc_gather(feat, rel_emb, norm_emb, head_batched,
                             rel_batched, tail_batched, time_batched)
    return _tc_score(h, r, t, nv).reshape(-1)
